# Initial kernel scaffold; baseline (speedup 1.0000x reference)
import jax
import jax.numpy as jnp
from jax.experimental import pallas as pl


def kernel(x, conv1_wmat, conv1_scale, conv1_bias, l0b0_c1_wmat, l0b0_c1_scale, l0b0_c1_bias, l0b0_c2_wmat, l0b0_c2_scale, l0b0_c2_bias, l0b1_c1_wmat, l0b1_c1_scale, l0b1_c1_bias, l0b1_c2_wmat, l0b1_c2_scale, l0b1_c2_bias, l1b0_c1_wmat, l1b0_c1_scale, l1b0_c1_bias, l1b0_c2_wmat, l1b0_c2_scale, l1b0_c2_bias, l1b0_ds_wmat, l1b0_ds_scale, l1b0_ds_bias, l1b1_c1_wmat, l1b1_c1_scale, l1b1_c1_bias, l1b1_c2_wmat, l1b1_c2_scale, l1b1_c2_bias, l2b0_c1_wmat, l2b0_c1_scale, l2b0_c1_bias, l2b0_c2_wmat, l2b0_c2_scale, l2b0_c2_bias, l2b0_ds_wmat, l2b0_ds_scale, l2b0_ds_bias, l2b1_c1_wmat, l2b1_c1_scale, l2b1_c1_bias, l2b1_c2_wmat, l2b1_c2_scale, l2b1_c2_bias, l3b0_c1_wmat, l3b0_c1_scale, l3b0_c1_bias, l3b0_c2_wmat, l3b0_c2_scale, l3b0_c2_bias, l3b0_ds_wmat, l3b0_ds_scale, l3b0_ds_bias, l3b1_c1_wmat, l3b1_c1_scale, l3b1_c1_bias, l3b1_c2_wmat, l3b1_c2_scale, l3b1_c2_bias, fc_w, fc_b):
    raise NotImplementedError("write your pallas kernel here")


# speedup vs baseline: 1.7411x; 1.7411x over previous
import functools

import jax
import jax.numpy as jnp
from jax.experimental import pallas as pl
from jax.experimental.pallas import tpu as pltpu

_BF = jnp.bfloat16
_F32 = jnp.float32


def _full_spec(shape):
    nd = len(shape)
    return pl.BlockSpec(shape, lambda i, _nd=nd: (0,) * _nd)


def _taps3():
    return [(di, dj) for di in range(3) for dj in range(3)]


def _stem_pool_body(p_ref, w_ref, s_ref, b_ref, o_ref):
    y = jnp.dot(p_ref[0], w_ref[...], preferred_element_type=_F32)
    y = y * s_ref[...] + b_ref[...]
    y = jnp.maximum(y, 0.0).astype(_BF)
    yr = y.reshape(112, 2, 56, 64)
    ev, od = yr[:, 0], yr[:, 1]
    aw = jnp.maximum(ev, od)
    osw = jnp.concatenate([jnp.zeros((112, 1, 64), _BF), od[:, :-1, :]], axis=1)
    wz = jnp.maximum(aw, osw)

    hh = wz.reshape(56, 2, 56, 64)
    a = jnp.maximum(hh[:, 0], hh[:, 1])
    o = hh[:, 1]
    os_ = jnp.concatenate([jnp.zeros((1, 56, 64), _BF), o[:-1]], axis=0)
    res = jnp.maximum(a, os_).reshape(1, 56, 56, 64)

    o_ref[...] = jnp.zeros_like(o_ref)
    o_ref[:, 1:57, 1:57, :] = res


def _stem_pool(patches, w, s, b):
    bsz = patches.shape[0]
    out_shape = jax.ShapeDtypeStruct((bsz, 58, 58, 64), _BF)
    return pl.pallas_call(
        _stem_pool_body,
        out_shape=out_shape,
        grid=(bsz,),
        in_specs=[
            pl.BlockSpec((1,) + patches.shape[1:], lambda i: (i, 0, 0)),
            _full_spec(w.shape),
            _full_spec(s.shape),
            _full_spec(b.shape),
        ],
        out_specs=pl.BlockSpec((1, 58, 58, 64), lambda i: (i, 0, 0, 0)),
        compiler_params=pltpu.CompilerParams(dimension_semantics=("parallel",)),
    )(patches, w, s, b)


def _conv3s1_body(*refs, bc, ho, wo, cin, n, res, head, concat):
    x_ref, w_ref, s_ref, b_ref = refs[:4]
    idx = 4
    r_ref = None
    if res is not None:
        r_ref = refs[idx]
        idx += 1
    if head:
        fcw_ref, fcb_ref = refs[idx], refs[idx + 1]
        idx += 2
    o_ref = refs[idx]

    m = bc * ho * wo
    if concat:
        aa = jnp.concatenate(
            [x_ref[:, di:di + ho, dj:dj + wo, :].reshape(m, cin)
             for di, dj in _taps3()], axis=1)
        acc = jnp.dot(aa, w_ref[...], preferred_element_type=_F32)
    else:
        acc = jnp.zeros((m, n), _F32)
        for t, (di, dj) in enumerate(_taps3()):
            a = x_ref[:, di:di + ho, dj:dj + wo, :].reshape(m, cin)
            acc = acc + jnp.dot(a, w_ref[t * cin:(t + 1) * cin, :],
                                preferred_element_type=_F32)

    y = acc * s_ref[...] + b_ref[...]
    if res == "padded":
        y = y + r_ref[:, 1:1 + ho, 1:1 + wo, :].reshape(m, n).astype(_F32)
    elif res == "flat":
        y = y + r_ref[...].reshape(m, n).astype(_F32)
    y = jnp.maximum(y, 0.0).astype(_BF)

    if head:
        feat = y.astype(_F32).reshape(bc, ho * wo, n).sum(axis=1) * (1.0 / (ho * wo))
        o_ref[...] = (jnp.dot(feat.astype(_BF), fcw_ref[...],
                              preferred_element_type=_F32) + fcb_ref[...])
    else:
        o_ref[...] = jnp.zeros_like(o_ref)
        o_ref[:, 1:1 + ho, 1:1 + wo, :] = y.reshape(bc, ho, wo, n)


def _conv3s1(xp, w, s, b, *, bc, res=None, r=None, head=False, fcw=None,
             fcb=None, concat=False):
    bsz, hp, wp, cin = xp.shape
    ho, wo = hp - 2, wp - 2
    n = w.shape[1]
    grid = (bsz // bc,)
    ins = [xp, w, s, b]
    specs = [
        pl.BlockSpec((bc, hp, wp, cin), lambda i: (i, 0, 0, 0)),
        _full_spec(w.shape), _full_spec(s.shape), _full_spec(b.shape),
    ]
    if res == "padded":
        ins.append(r)
        specs.append(pl.BlockSpec((bc, hp, wp, n), lambda i: (i, 0, 0, 0)))
    elif res == "flat":
        ins.append(r)
        specs.append(pl.BlockSpec((bc, ho, wo, n), lambda i: (i, 0, 0, 0)))
    if head:
        ins += [fcw, fcb]
        specs += [_full_spec(fcw.shape), _full_spec(fcb.shape)]
        out_shape = jax.ShapeDtypeStruct((bsz, fcw.shape[1]), _F32)
        out_spec = pl.BlockSpec((bc, fcw.shape[1]), lambda i: (i, 0))
    else:
        out_shape = jax.ShapeDtypeStruct((bsz, hp, wp, n), _BF)
        out_spec = pl.BlockSpec((bc, hp, wp, n), lambda i: (i, 0, 0, 0))
    body = functools.partial(_conv3s1_body, bc=bc, ho=ho, wo=wo, cin=cin, n=n,
                             res=res, head=head, concat=concat)
    return pl.pallas_call(
        body,
        out_shape=out_shape,
        grid=grid,
        in_specs=specs,
        out_specs=out_spec,
        compiler_params=pltpu.CompilerParams(dimension_semantics=("parallel",)),
    )(*ins)


def _conv3s2_ds_body(p00, p01, p10, p11, w_ref, s_ref, b_ref,
                     dw_ref, dss_ref, dsb_ref, o1_ref, o2_ref,
                     *, bc, ho, wo, cin, n, concat):
    phases = (p00, p01, p10, p11)
    m = bc * ho * wo

    def tap(di, dj):
        p = phases[(di % 2) * 2 + (dj % 2)]
        oi, oj = di // 2, dj // 2
        return p[:, oi:oi + ho, oj:oj + wo, :].reshape(m, cin)

    if concat:
        aa = jnp.concatenate([tap(di, dj) for di, dj in _taps3()], axis=1)
        acc = jnp.dot(aa, w_ref[...], preferred_element_type=_F32)
    else:
        acc = jnp.zeros((m, n), _F32)
        for t, (di, dj) in enumerate(_taps3()):
            acc = acc + jnp.dot(tap(di, dj), w_ref[t * cin:(t + 1) * cin, :],
                                preferred_element_type=_F32)
    y = jnp.maximum(acc * s_ref[...] + b_ref[...], 0.0).astype(_BF)
    o1_ref[...] = jnp.zeros_like(o1_ref)
    o1_ref[:, 1:1 + ho, 1:1 + wo, :] = y.reshape(bc, ho, wo, n)

    a = p11[:, 0:ho, 0:wo, :].reshape(m, cin)
    idn = jnp.dot(a, dw_ref[...], preferred_element_type=_F32)
    idn = idn * dss_ref[...] + dsb_ref[...]
    o2_ref[...] = idn.astype(_BF).reshape(bc, ho, wo, dw_ref.shape[1])


def _conv3s2_ds(phases, w, s, b, dw, dss, dsb, *, bc, concat=False):
    bsz, hp, wp, cin = phases[0].shape
    ho, wo = hp - 1, wp - 1
    n = w.shape[1]
    grid = (bsz // bc,)
    pspec = pl.BlockSpec((bc, hp, wp, cin), lambda i: (i, 0, 0, 0))
    body = functools.partial(_conv3s2_ds_body, bc=bc, ho=ho, wo=wo, cin=cin,
                             n=n, concat=concat)
    return pl.pallas_call(
        body,
        out_shape=(
            jax.ShapeDtypeStruct((bsz, ho + 2, wo + 2, n), _BF),
            jax.ShapeDtypeStruct((bsz, ho, wo, n), _BF),
        ),
        grid=grid,
        in_specs=[pspec, pspec, pspec, pspec,
                  _full_spec(w.shape), _full_spec(s.shape), _full_spec(b.shape),
                  _full_spec(dw.shape), _full_spec(dss.shape),
                  _full_spec(dsb.shape)],
        out_specs=(
            pl.BlockSpec((bc, ho + 2, wo + 2, n), lambda i: (i, 0, 0, 0)),
            pl.BlockSpec((bc, ho, wo, n), lambda i: (i, 0, 0, 0)),
        ),
        compiler_params=pltpu.CompilerParams(dimension_semantics=("parallel",)),
    )(*phases, w, s, b, dw, dss, dsb)


def _stem_patches(x):
    bsz = x.shape[0]
    xh = jnp.transpose(x, (0, 2, 3, 1)).astype(_BF)
    xp = jnp.pad(xh, ((0, 0), (3, 3), (3, 3), (0, 0)))
    cols = [xp[:, di:di + 224:2, dj:dj + 224:2, :]
            for di in range(7) for dj in range(7)]
    pat = jnp.stack(cols, axis=3)
    pat = jnp.concatenate([pat[:, :, 0::2], pat[:, :, 1::2]], axis=2)
    return pat.reshape(bsz, 112 * 112, 7 * 7 * 3)


def _phase_split(xp):
    return tuple(xp[:, a::2, b::2, :] for a in range(2) for b in range(2))


def _chunk(bsz, want):
    c = min(want, bsz)
    while bsz % c:
        c -= 1
    return c


def kernel(x, conv1_wmat, conv1_scale, conv1_bias, l0b0_c1_wmat, l0b0_c1_scale, l0b0_c1_bias, l0b0_c2_wmat, l0b0_c2_scale, l0b0_c2_bias, l0b1_c1_wmat, l0b1_c1_scale, l0b1_c1_bias, l0b1_c2_wmat, l0b1_c2_scale, l0b1_c2_bias, l1b0_c1_wmat, l1b0_c1_scale, l1b0_c1_bias, l1b0_c2_wmat, l1b0_c2_scale, l1b0_c2_bias, l1b0_ds_wmat, l1b0_ds_scale, l1b0_ds_bias, l1b1_c1_wmat, l1b1_c1_scale, l1b1_c1_bias, l1b1_c2_wmat, l1b1_c2_scale, l1b1_c2_bias, l2b0_c1_wmat, l2b0_c1_scale, l2b0_c1_bias, l2b0_c2_wmat, l2b0_c2_scale, l2b0_c2_bias, l2b0_ds_wmat, l2b0_ds_scale, l2b0_ds_bias, l2b1_c1_wmat, l2b1_c1_scale, l2b1_c1_bias, l2b1_c2_wmat, l2b1_c2_scale, l2b1_c2_bias, l3b0_c1_wmat, l3b0_c1_scale, l3b0_c1_bias, l3b0_c2_wmat, l3b0_c2_scale, l3b0_c2_bias, l3b0_ds_wmat, l3b0_ds_scale, l3b0_ds_bias, l3b1_c1_wmat, l3b1_c1_scale, l3b1_c1_bias, l3b1_c2_wmat, l3b1_c2_scale, l3b1_c2_bias, fc_w, fc_b):
    x = x.reshape(-1, 3, 224, 224)
    bsz = x.shape[0]
    bc1 = _chunk(bsz, 1)
    bc2 = _chunk(bsz, 4)
    bc3 = _chunk(bsz, 16)
    bc4 = _chunk(bsz, 32)

    p1 = _stem_pool(_stem_patches(x), conv1_wmat, conv1_scale, conv1_bias)

    y = _conv3s1(p1, l0b0_c1_wmat, l0b0_c1_scale, l0b0_c1_bias, bc=bc1)
    p2 = _conv3s1(y, l0b0_c2_wmat, l0b0_c2_scale, l0b0_c2_bias, bc=bc1,
                  res="padded", r=p1)
    y = _conv3s1(p2, l0b1_c1_wmat, l0b1_c1_scale, l0b1_c1_bias, bc=bc1)
    p3 = _conv3s1(y, l0b1_c2_wmat, l0b1_c2_scale, l0b1_c2_bias, bc=bc1,
                  res="padded", r=p2)

    y1p, idn = _conv3s2_ds(_phase_split(p3), l1b0_c1_wmat, l1b0_c1_scale,
                           l1b0_c1_bias, l1b0_ds_wmat, l1b0_ds_scale,
                           l1b0_ds_bias, bc=bc2)
    p4 = _conv3s1(y1p, l1b0_c2_wmat, l1b0_c2_scale, l1b0_c2_bias, bc=bc2,
                  res="flat", r=idn)
    y = _conv3s1(p4, l1b1_c1_wmat, l1b1_c1_scale, l1b1_c1_bias, bc=bc2)
    p5 = _conv3s1(y, l1b1_c2_wmat, l1b1_c2_scale, l1b1_c2_bias, bc=bc2,
                  res="padded", r=p4)

    y1p, idn = _conv3s2_ds(_phase_split(p5), l2b0_c1_wmat, l2b0_c1_scale,
                           l2b0_c1_bias, l2b0_ds_wmat, l2b0_ds_scale,
                           l2b0_ds_bias, bc=bc3)
    p6 = _conv3s1(y1p, l2b0_c2_wmat, l2b0_c2_scale, l2b0_c2_bias, bc=bc3,
                  res="flat", r=idn)
    y = _conv3s1(p6, l2b1_c1_wmat, l2b1_c1_scale, l2b1_c1_bias, bc=bc3)
    p7 = _conv3s1(y, l2b1_c2_wmat, l2b1_c2_scale, l2b1_c2_bias, bc=bc3,
                  res="padded", r=p6)

    y1p, idn = _conv3s2_ds(_phase_split(p7), l3b0_c1_wmat, l3b0_c1_scale,
                           l3b0_c1_bias, l3b0_ds_wmat, l3b0_ds_scale,
                           l3b0_ds_bias, bc=bc4)
    p8 = _conv3s1(y1p, l3b0_c2_wmat, l3b0_c2_scale, l3b0_c2_bias, bc=bc4,
                  res="flat", r=idn)
    y = _conv3s1(p8, l3b1_c1_wmat, l3b1_c1_scale, l3b1_c1_bias, bc=bc4)
    return _conv3s1(y, l3b1_c2_wmat, l3b1_c2_scale, l3b1_c2_bias, bc=bc4,
                    res="padded", r=p8, head=True, fcw=fc_w, fcb=fc_b)
